# baseline (device time: 16423 ns/iter reference)
import jax
import jax.numpy as jnp
from jax import lax
from jax.experimental import pallas as pl
from jax.experimental.pallas import tpu as pltpu

C = 8


def kernel(x):
    m, n = x.shape
    half = m // 2
    ch = half // C

    def body(x_ref, out_ref, send_ref, xrecv_ref,
             xs_sems, xr_sems, zs_sems, zr_sems):
        my_x = lax.axis_index("x")
        my_y = lax.axis_index("y")
        my_z = lax.axis_index("z")
        xpeer = (1 - my_x, my_y, my_z)
        zpeer = (my_x, my_y, 1 - my_z)
        base = my_z * half

        barrier_sem = pltpu.get_barrier_semaphore()
        for nbr in (xpeer, zpeer):
            pl.semaphore_signal(
                barrier_sem, inc=1, device_id=nbr,
                device_id_type=pl.DeviceIdType.MESH,
            )
        send_ref[...] = x_ref[pl.ds(base, half), :].astype(jnp.bfloat16)
        pl.semaphore_wait(barrier_sem, 2)

        x_rdmas = []
        for c in range(C):
            sl = pl.ds(c * ch, ch)
            r = pltpu.make_async_remote_copy(
                src_ref=send_ref.at[sl],
                dst_ref=xrecv_ref.at[sl],
                send_sem=xs_sems.at[c],
                recv_sem=xr_sems.at[c],
                device_id=xpeer,
                device_id_type=pl.DeviceIdType.MESH,
            )
            r.start()
            x_rdmas.append(r)

        z_rdmas = []
        for c in range(C):
            osl = pl.ds(base + c * ch, ch)
            x_rdmas[c].wait_recv()
            out_ref[osl, :] = (
                x_ref[osl, :].astype(jnp.bfloat16)
                + xrecv_ref[pl.ds(c * ch, ch), :])
            r = pltpu.make_async_remote_copy(
                src_ref=out_ref.at[osl],
                dst_ref=out_ref.at[osl],
                send_sem=zs_sems.at[c],
                recv_sem=zr_sems.at[c],
                device_id=zpeer,
                device_id_type=pl.DeviceIdType.MESH,
            )
            r.start()
            z_rdmas.append(r)

        for c in range(C):
            z_rdmas[c].wait_recv()

        for c in range(C):
            x_rdmas[c].wait_send()
            z_rdmas[c].wait_send()

    return pl.pallas_call(
        body,
        out_shape=jax.ShapeDtypeStruct((m, n), jnp.bfloat16),
        in_specs=[pl.BlockSpec(memory_space=pltpu.VMEM)],
        out_specs=pl.BlockSpec(memory_space=pltpu.VMEM),
        scratch_shapes=[
            pltpu.VMEM((half, n), jnp.bfloat16),
            pltpu.VMEM((half, n), jnp.bfloat16),
            pltpu.SemaphoreType.DMA((C,)),
            pltpu.SemaphoreType.DMA((C,)),
            pltpu.SemaphoreType.DMA((C,)),
            pltpu.SemaphoreType.DMA((C,)),
        ],
        compiler_params=pltpu.CompilerParams(collective_id=0),
    )(x)


# device time: 11932 ns/iter; 1.3764x vs baseline; 1.3764x over previous
import jax
import jax.numpy as jnp
from jax import lax
from jax.experimental import pallas as pl
from jax.experimental.pallas import tpu as pltpu

C = 8


def kernel(x):
    m, n = x.shape
    half = m // 2
    ch = half // C

    def body(x_hbm_ref, out_ref, xloc_ref, send_ref, xrecv_ref, copy_sem,
             xs_sems, xr_sems, zs_sems, zr_sems):
        my_x = lax.axis_index("x")
        my_y = lax.axis_index("y")
        my_z = lax.axis_index("z")
        xpeer = (1 - my_x, my_y, my_z)
        zpeer = (my_x, my_y, 1 - my_z)
        base = my_z * half

        cp = pltpu.make_async_copy(
            x_hbm_ref.at[pl.ds(base, half)], xloc_ref, copy_sem)
        cp.start()

        barrier_sem = pltpu.get_barrier_semaphore()
        for nbr in (xpeer, zpeer):
            pl.semaphore_signal(
                barrier_sem, inc=1, device_id=nbr,
                device_id_type=pl.DeviceIdType.MESH,
            )
        cp.wait()
        send_ref[...] = xloc_ref[...].astype(jnp.bfloat16)
        pl.semaphore_wait(barrier_sem, 2)

        x_rdmas = []
        for c in range(C):
            sl = pl.ds(c * ch, ch)
            r = pltpu.make_async_remote_copy(
                src_ref=send_ref.at[sl],
                dst_ref=xrecv_ref.at[sl],
                send_sem=xs_sems.at[c],
                recv_sem=xr_sems.at[c],
                device_id=xpeer,
                device_id_type=pl.DeviceIdType.MESH,
            )
            r.start()
            x_rdmas.append(r)

        z_rdmas = []
        for c in range(C):
            sl = pl.ds(c * ch, ch)
            osl = pl.ds(base + c * ch, ch)
            x_rdmas[c].wait_recv()
            out_ref[osl, :] = (
                xloc_ref[sl, :].astype(jnp.bfloat16) + xrecv_ref[sl, :])
            r = pltpu.make_async_remote_copy(
                src_ref=out_ref.at[osl],
                dst_ref=out_ref.at[osl],
                send_sem=zs_sems.at[c],
                recv_sem=zr_sems.at[c],
                device_id=zpeer,
                device_id_type=pl.DeviceIdType.MESH,
            )
            r.start()
            z_rdmas.append(r)

        for c in range(C):
            z_rdmas[c].wait_recv()

        for c in range(C):
            x_rdmas[c].wait_send()
            z_rdmas[c].wait_send()

    return pl.pallas_call(
        body,
        out_shape=jax.ShapeDtypeStruct((m, n), jnp.bfloat16),
        in_specs=[pl.BlockSpec(memory_space=pltpu.MemorySpace.HBM)],
        out_specs=pl.BlockSpec(memory_space=pltpu.VMEM),
        scratch_shapes=[
            pltpu.VMEM((half, n), jnp.float32),
            pltpu.VMEM((half, n), jnp.bfloat16),
            pltpu.VMEM((half, n), jnp.bfloat16),
            pltpu.SemaphoreType.DMA,
            pltpu.SemaphoreType.DMA((C,)),
            pltpu.SemaphoreType.DMA((C,)),
            pltpu.SemaphoreType.DMA((C,)),
            pltpu.SemaphoreType.DMA((C,)),
        ],
        compiler_params=pltpu.CompilerParams(collective_id=0),
    )(x)
